# baseline (speedup 1.0000x reference)
import jax
import jax.numpy as jnp
import numpy as np
from jax import lax
from jax.experimental import pallas as pl
from jax.experimental.pallas import tpu as pltpu
from jax.experimental.pallas import tpu_sc as plsc

VOCAB = 1_000_000
D = 64
B = 4096
L = 200
BL = B * L
V2 = (VOCAB // 128) * 64 + 64

NC = 2
NS = 16
NW = NC * NS
BC = B // NW
LANES = 16
NBC = BC // LANES
SCALE = 8.0

BN = 2048


def _positional_encoding() -> np.ndarray:
    depth = D / 2
    positions = np.arange(L)[:, np.newaxis]
    depths = np.arange(depth)[np.newaxis, :] / depth
    angle_rates = 1 / 10000**depths
    angle_rads = positions * angle_rates
    pe = np.concatenate([np.sin(angle_rads), np.cos(angle_rads)], axis=-1)
    return pe.astype(np.float32)


_PE = _positional_encoding().reshape(-1)


def _tbody(a_ref, o_ref):
    t = a_ref[...].T * SCALE
    for h in range(BN // 128):
        o_ref[h * 64:(h + 1) * 64, :] = jnp.concatenate(
            [t[h * 128:h * 128 + 64, :], t[h * 128 + 64:h * 128 + 128, :]],
            axis=1)


def _body(t2_hbm, xT_hbm, pe_hbm, y_hbm, xidx_v, lines_v, pe_v, rows_a,
          rows_b, out_a, out_b, sem_a, sem_b, wsem_a, wsem_b):
    wid = lax.axis_index("s") * NC + lax.axis_index("c")
    b0 = wid * BC

    pltpu.sync_copy(xT_hbm.at[:, pl.ds(b0, BC)], xidx_v)
    pltpu.sync_copy(pe_hbm, pe_v)

    def prep(l, carry):
        for k in range(NBC):
            sl = pl.ds(k * LANES, LANES)
            i = xidx_v[l, sl]
            lines_v[l, sl] = lax.shift_left(
                lax.shift_right_logical(i, 7), 6) | (i & 63)
        return carry

    lax.fori_loop(0, L, prep, 0, unroll=4)

    def fire(l, buf, sem):
        pltpu.make_async_copy(t2_hbm.at[lines_v.at[l]], buf, sem).start()

    def wait(l, buf, sem):
        pltpu.make_async_copy(t2_hbm.at[lines_v.at[l]], buf, sem).wait()

    iotas = [lax.iota(jnp.int32, 16) + (k * LANES) for k in range(NBC)]

    def compute(l, buf, obuf):
        halves = [lax.shift_left(
            lax.shift_right_logical(xidx_v[l, pl.ds(k * LANES, LANES)], 6)
            & 1, 6) for k in range(NBC)]

        pe_ls = [pe_v[pl.ds(l * D + jg * LANES, LANES)]
                 for jg in range(D // LANES)]
        iota16 = lax.iota(jnp.int32, 16)

        @plsc.parallel_loop(0, LANES, unroll=4)
        def _(j0):
            perm = (j0 + iota16) & (LANES - 1)
            for jg in range(D // LANES):
                joff = jg * LANES + perm
                pe16 = pe_ls[jg][perm]
                for k in range(NBC):
                    v = plsc.load_gather(buf, [iotas[k], halves[k] + joff])
                    plsc.store_scatter(obuf, [joff, iotas[k]], v + pe16)

    def wfire(l, obuf, wsem):
        pltpu.make_async_copy(obuf, y_hbm.at[l, :, pl.ds(b0, BC)],
                              wsem).start()

    def wwait(l, obuf, wsem):
        pltpu.make_async_copy(obuf, y_hbm.at[l, :, pl.ds(b0, BC)],
                              wsem).wait()

    bufs = (rows_a, rows_b)
    sems = (sem_a, sem_b)
    obufs = (out_a, out_b)
    wsems = (wsem_a, wsem_b)
    fire(0, rows_a, sem_a)

    def step(i, carry):
        for p in range(2):
            l = i * 2 + p
            cur, nxt = bufs[p], bufs[1 - p]
            scur, snxt = sems[p], sems[1 - p]
            wait(l, cur, scur)

            @pl.when(l + 1 < L)
            def _():
                fire(l + 1, nxt, snxt)

            @pl.when(l >= 2)
            def _():
                wwait(l, obufs[p], wsems[p])

            compute(l, cur, obufs[p])
            wfire(l, obufs[p], wsems[p])
        return carry

    lax.fori_loop(0, L // 2, step, 0)
    wwait(L - 2, out_a, wsem_a)
    wwait(L - 1, out_b, wsem_b)


@jax.jit
def _run(xT, tableT, pe):
    t2 = pl.pallas_call(
        _tbody,
        grid=(VOCAB // BN + 1,),
        out_shape=jax.ShapeDtypeStruct((V2, 2 * D), jnp.float32),
        in_specs=[pl.BlockSpec((D, BN), lambda i: (0, i))],
        out_specs=pl.BlockSpec((BN // 2, 2 * D), lambda i: (i, 0)),
    )(tableT)

    kern = pl.kernel(
        _body,
        out_type=jax.ShapeDtypeStruct((L, D, B), jnp.float32),
        mesh=plsc.VectorSubcoreMesh(core_axis_name="c", subcore_axis_name="s"),
        compiler_params=pltpu.CompilerParams(needs_layout_passes=False),
        scratch_types=[
            pltpu.VMEM((L, BC), jnp.int32),
            pltpu.VMEM((L, BC), jnp.int32),
            pltpu.VMEM((L * D,), jnp.float32),
            pltpu.VMEM((BC, 2 * D), jnp.float32),
            pltpu.VMEM((BC, 2 * D), jnp.float32),
            pltpu.VMEM((D, BC), jnp.float32),
            pltpu.VMEM((D, BC), jnp.float32),
            pltpu.SemaphoreType.DMA,
            pltpu.SemaphoreType.DMA,
            pltpu.SemaphoreType.DMA,
            pltpu.SemaphoreType.DMA,
        ],
    )
    return kern(t2, xT, pe)


def kernel(x, table):
    xT = jnp.swapaxes(x, 0, 1).astype(jnp.int32)
    tableT = jnp.swapaxes(table, 0, 1)
    y = _run(xT, tableT, jnp.asarray(_PE))
    return jnp.transpose(y, (2, 0, 1))


# speedup vs baseline: 1.4903x; 1.4903x over previous
import jax
import jax.numpy as jnp
import numpy as np
from jax import lax
from jax.experimental import pallas as pl
from jax.experimental.pallas import tpu as pltpu
from jax.experimental.pallas import tpu_sc as plsc

VOCAB = 1_000_000
D = 64
B = 4096
L = 200
BL = B * L
V2 = (VOCAB // 128) * 64 + 64

NC = 2
NS = 16
NW = NC * NS
BC = B // NW
LANES = 16
NBC = BC // LANES
SCALE = 8.0

BN = 4096


def _positional_encoding() -> np.ndarray:
    depth = D / 2
    positions = np.arange(L)[:, np.newaxis]
    depths = np.arange(depth)[np.newaxis, :] / depth
    angle_rates = 1 / 10000**depths
    angle_rads = positions * angle_rates
    pe = np.concatenate([np.sin(angle_rads), np.cos(angle_rads)], axis=-1)
    return pe.astype(np.float32)


_PE = _positional_encoding().reshape(-1)


def _tbody(a_ref, o_ref):
    t = a_ref[...].T * SCALE
    for h in range(BN // 128):
        o_ref[h * 64:(h + 1) * 64, :] = jnp.concatenate(
            [t[h * 128:h * 128 + 64, :], t[h * 128 + 64:h * 128 + 128, :]],
            axis=1)


def _body(t2_hbm, xT_hbm, pe_hbm, y_hbm, xidx_v, lines_v, pe_v, rows_a,
          rows_b, out_a, out_b, sem_a, sem_b, wsem_a, wsem_b):
    wid = lax.axis_index("s") * NC + lax.axis_index("c")
    b0 = wid * BC

    pltpu.sync_copy(xT_hbm.at[:, pl.ds(b0, BC)], xidx_v)
    pltpu.sync_copy(pe_hbm, pe_v)

    def prep(l, carry):
        for k in range(NBC):
            sl = pl.ds(k * LANES, LANES)
            i = xidx_v[l, sl]
            lines_v[l, sl] = lax.shift_left(
                lax.shift_right_logical(i, 7), 6) | (i & 63)
        return carry

    lax.fori_loop(0, L, prep, 0, unroll=4)

    def fire(l, buf, sem):
        pltpu.make_async_copy(t2_hbm.at[lines_v.at[l]], buf, sem).start()

    def wait(l, buf, sem):
        pltpu.make_async_copy(t2_hbm.at[lines_v.at[l]], buf, sem).wait()

    iotas = [lax.iota(jnp.int32, 16) + (k * LANES) for k in range(NBC)]

    def compute(l, buf, obuf):
        halves = [lax.shift_left(
            lax.shift_right_logical(xidx_v[l, pl.ds(k * LANES, LANES)], 6)
            & 1, 6) for k in range(NBC)]

        pe_ls = [pe_v[pl.ds(l * D + jg * LANES, LANES)]
                 for jg in range(D // LANES)]
        iota16 = lax.iota(jnp.int32, 16)

        @plsc.parallel_loop(0, LANES, unroll=4)
        def _(j0):
            perm = (j0 + iota16) & (LANES - 1)
            for jg in range(D // LANES):
                joff = jg * LANES + perm
                pe16 = pe_ls[jg][perm]
                for k in range(NBC):
                    v = plsc.load_gather(buf, [iotas[k], halves[k] + joff])
                    plsc.store_scatter(obuf, [joff, iotas[k]], v + pe16)

    def wfire(l, obuf, wsem):
        pltpu.make_async_copy(obuf, y_hbm.at[l, :, pl.ds(b0, BC)],
                              wsem).start()

    def wwait(l, obuf, wsem):
        pltpu.make_async_copy(obuf, y_hbm.at[l, :, pl.ds(b0, BC)],
                              wsem).wait()

    bufs = (rows_a, rows_b)
    sems = (sem_a, sem_b)
    obufs = (out_a, out_b)
    wsems = (wsem_a, wsem_b)
    fire(0, rows_a, sem_a)

    def step(i, carry):
        for p in range(2):
            l = i * 2 + p
            cur, nxt = bufs[p], bufs[1 - p]
            scur, snxt = sems[p], sems[1 - p]
            wait(l, cur, scur)

            @pl.when(l + 1 < L)
            def _():
                fire(l + 1, nxt, snxt)

            @pl.when(l >= 2)
            def _():
                wwait(l, obufs[p], wsems[p])

            compute(l, cur, obufs[p])
            wfire(l, obufs[p], wsems[p])
        return carry

    lax.fori_loop(0, L // 2, step, 0)
    wwait(L - 2, out_a, wsem_a)
    wwait(L - 1, out_b, wsem_b)


@jax.jit
def _run(xT, tableT, pe):
    t2 = pl.pallas_call(
        _tbody,
        grid=(VOCAB // BN + 1,),
        out_shape=jax.ShapeDtypeStruct((V2, 2 * D), jnp.float32),
        in_specs=[pl.BlockSpec((D, BN), lambda i: (0, i))],
        out_specs=pl.BlockSpec((BN // 2, 2 * D), lambda i: (i, 0)),
    )(tableT)

    kern = pl.kernel(
        _body,
        out_type=jax.ShapeDtypeStruct((L, D, B), jnp.float32),
        mesh=plsc.VectorSubcoreMesh(core_axis_name="c", subcore_axis_name="s"),
        compiler_params=pltpu.CompilerParams(needs_layout_passes=False),
        scratch_types=[
            pltpu.VMEM((L, BC), jnp.int32),
            pltpu.VMEM((L, BC), jnp.int32),
            pltpu.VMEM((L * D,), jnp.float32),
            pltpu.VMEM((BC, 2 * D), jnp.float32),
            pltpu.VMEM((BC, 2 * D), jnp.float32),
            pltpu.VMEM((D, BC), jnp.float32),
            pltpu.VMEM((D, BC), jnp.float32),
            pltpu.SemaphoreType.DMA,
            pltpu.SemaphoreType.DMA,
            pltpu.SemaphoreType.DMA,
            pltpu.SemaphoreType.DMA,
        ],
    )
    return kern(t2, xT, pe)


def kernel(x, table):
    xT = jnp.swapaxes(x, 0, 1).astype(jnp.int32)
    tableT = jnp.swapaxes(table, 0, 1)
    y = _run(xT, tableT, jnp.asarray(_PE))
    return jnp.transpose(y, (2, 0, 1))
